# baseline (speedup 1.0000x reference)
import jax
import jax.numpy as jnp
from jax.experimental import pallas as pl
from jax.experimental.pallas import tpu as pltpu

_FLIP = 2**31 - 1


def _orderable(x):
    u = jax.lax.bitcast_convert_type(x, jnp.int32)
    return jnp.where(u >= 0, u, u ^ _FLIP)


def _sum0(x):
    n = x.shape[0]
    while n > 1:
        h = n // 2
        x = x[:h] + x[h:]
        n = h
    return x


def _mask_body(zt_ref, *refs):
    in_refs = refs[:16]
    out_refs = refs[16:]
    zt = zt_ref[...]
    r = zt.shape[1]
    keys = []
    his = []
    los = []
    one16s = []
    zero16s = []
    ks = []
    for li in range(4):
        W1t = in_refs[4 * li][...]
        b1 = in_refs[4 * li + 1][...]
        W2t = in_refs[4 * li + 2][...]
        b2 = in_refs[4 * li + 3][...]
        ht = jnp.maximum(
            jnp.dot(W1t, zt, preferred_element_type=jnp.float32) + b1, 0.0)
        logits_t = jnp.dot(W2t, ht, preferred_element_type=jnp.float32) + b2
        key = _orderable(logits_t)
        keys.append(key)
        his.append((key >> 16).astype(jnp.int16))
        los.append(key.astype(jnp.int16) ^ (-0x8000))
        ks.append(max(1, logits_t.shape[0] // 2))

    def step_hi(i, ps):
        bit = jnp.broadcast_to(
            jnp.left_shift(jnp.int32(1), jnp.int32(15) - i), (1, r)
        ).astype(jnp.int16)
        out = []
        for hi, k, p in zip(his, ks, ps):
            cand = p | bit
            candb = jnp.broadcast_to(cand ^ (-0x8000), hi.shape)
            cnt = _sum0(jnp.where(hi >= candb, jnp.int16(1), jnp.int16(0)))
            out.append(jnp.where(cnt >= k, cand, p))
        return tuple(out)

    zero16 = tuple(jnp.zeros((1, r), jnp.int16) for _ in range(4))
    ps = jax.lax.fori_loop(0, 16, step_hi, zero16, unroll=8)

    t_his, ms, lims = [], [], []
    for hi, lo, k, p in zip(his, los, ks, ps):
        t_hi = p ^ (-0x8000)
        t_hib = jnp.broadcast_to(t_hi, hi.shape)
        n_gt = _sum0(jnp.where(hi > t_hib, jnp.int16(1), jnp.int16(0)))
        t_his.append(t_hi)
        ms.append(jnp.int16(k) - n_gt)
        lims.append(jnp.where(hi == t_hib, lo, jnp.int16(-0x8000)))

    def step_lo(i, ps2):
        bit = jnp.broadcast_to(
            jnp.left_shift(jnp.int32(1), jnp.int32(15) - i), (1, r)
        ).astype(jnp.int16)
        out = []
        for lom, m, p in zip(lims, ms, ps2):
            cand = p | bit
            candb = jnp.broadcast_to(cand ^ (-0x8000), lom.shape)
            cnt = _sum0(jnp.where(lom >= candb, jnp.int16(1), jnp.int16(0)))
            out.append(jnp.where(cnt >= m, cand, p))
        return tuple(out)

    ps2 = jax.lax.fori_loop(0, 16, step_lo, zero16, unroll=8)

    for li in range(4):
        t_hi32 = (t_his[li]).astype(jnp.int32)
        t_lo32 = (ps2[li] ^ (-0x8000)).astype(jnp.int32) ^ 0x8000
        thresh = (t_hi32 << 16) | (t_lo32 & 0xFFFF)
        mask_t = jnp.where(keys[li] >= thresh, 2.0, 0.0).astype(jnp.float32)
        out_refs[li][...] = mask_t.T


def kernel(z, W1_0, b1_0, W2_0, b2_0, W1_1, b1_1, W2_1, b2_1,
           W1_2, b1_2, W2_2, b2_2, W1_3, b1_3, W2_3, b2_3):
    B, Z = z.shape
    R = 1024 if B % 1024 == 0 else B
    grid = (B // R,)
    zt = z.T

    layers = [
        (W1_0, b1_0, W2_0, b2_0),
        (W1_1, b1_1, W2_1, b2_1),
        (W1_2, b1_2, W2_2, b2_2),
        (W1_3, b1_3, W2_3, b2_3),
    ]
    chans = [W2.shape[1] for (_, _, W2, _) in layers]

    in_arrays = [zt]
    in_specs = [pl.BlockSpec((Z, R), lambda i: (0, i))]
    for (W1, b1, W2, b2) in layers:
        for w in (W1.T, b1.reshape(-1, 1), W2.T, b2.reshape(-1, 1)):
            in_arrays.append(w)
            in_specs.append(pl.BlockSpec(w.shape, lambda i: (0, 0)))

    out_shape = [jax.ShapeDtypeStruct((B, c), jnp.float32) for c in chans]
    out_specs = [pl.BlockSpec((R, c), lambda i: (i, 0)) for c in chans]

    out = pl.pallas_call(
        _mask_body,
        grid=grid,
        in_specs=in_specs,
        out_specs=out_specs,
        out_shape=out_shape,
        compiler_params=pltpu.CompilerParams(
            dimension_semantics=("arbitrary",),
        ),
    )(*in_arrays)
    return tuple(out)


# speedup vs baseline: 1.0002x; 1.0002x over previous
import jax
import jax.numpy as jnp
from jax.experimental import pallas as pl
from jax.experimental.pallas import tpu as pltpu

_FLIP = 2**31 - 1


def _orderable(x):
    u = jax.lax.bitcast_convert_type(x, jnp.int32)
    return jnp.where(u >= 0, u, u ^ _FLIP)


def _sum0(x):
    n = x.shape[0]
    while n > 1:
        h = n // 2
        x = x[:h] + x[h:]
        n = h
    return x


def _mask_body(zt_ref, *refs):
    in_refs = refs[:16]
    out_refs = refs[16:]
    zt = zt_ref[...]
    r = zt.shape[1]
    keys = []
    his = []
    los = []
    one16s = []
    zero16s = []
    ks = []
    for li in range(4):
        W1t = in_refs[4 * li][...]
        b1 = in_refs[4 * li + 1][...]
        W2t = in_refs[4 * li + 2][...]
        b2 = in_refs[4 * li + 3][...]
        ht = jnp.maximum(
            jnp.dot(W1t, zt, preferred_element_type=jnp.float32) + b1, 0.0)
        logits_t = jnp.dot(W2t, ht, preferred_element_type=jnp.float32) + b2
        key = _orderable(logits_t)
        keys.append(key)
        his.append((key >> 16).astype(jnp.int16))
        los.append(key.astype(jnp.int16) ^ (-0x8000))
        ks.append(max(1, logits_t.shape[0] // 2))

    def step_hi(i, ps):
        bit = jnp.broadcast_to(
            jnp.left_shift(jnp.int32(1), jnp.int32(15) - i), (1, r)
        ).astype(jnp.int16)
        out = []
        for hi, k, p in zip(his, ks, ps):
            cand = p | bit
            candb = jnp.broadcast_to(cand ^ (-0x8000), hi.shape)
            cnt = _sum0(jnp.where(hi >= candb, jnp.int16(1), jnp.int16(0)))
            out.append(jnp.where(cnt >= k, cand, p))
        return tuple(out)

    zero16 = tuple(jnp.zeros((1, r), jnp.int16) for _ in range(4))
    ps = jax.lax.fori_loop(0, 16, step_hi, zero16, unroll=8)

    t_his, ms, lims = [], [], []
    for hi, lo, k, p in zip(his, los, ks, ps):
        t_hi = p ^ (-0x8000)
        t_hib = jnp.broadcast_to(t_hi, hi.shape)
        n_gt = _sum0(jnp.where(hi > t_hib, jnp.int16(1), jnp.int16(0)))
        t_his.append(t_hi)
        ms.append(jnp.int16(k) - n_gt)
        lims.append(jnp.where(hi == t_hib, lo, jnp.int16(-0x8000)))

    def step_lo(i, ps2):
        bit = jnp.broadcast_to(
            jnp.left_shift(jnp.int32(1), jnp.int32(15) - i), (1, r)
        ).astype(jnp.int16)
        out = []
        for lom, m, p in zip(lims, ms, ps2):
            cand = p | bit
            candb = jnp.broadcast_to(cand ^ (-0x8000), lom.shape)
            cnt = _sum0(jnp.where(lom >= candb, jnp.int16(1), jnp.int16(0)))
            out.append(jnp.where(cnt >= m, cand, p))
        return tuple(out)

    ps2 = jax.lax.fori_loop(0, 16, step_lo, zero16, unroll=8)

    for li in range(4):
        t_hi32 = (t_his[li]).astype(jnp.int32)
        t_lo32 = (ps2[li] ^ (-0x8000)).astype(jnp.int32) ^ 0x8000
        thresh = (t_hi32 << 16) | (t_lo32 & 0xFFFF)
        mask_t = jnp.where(keys[li] >= thresh, 2.0, 0.0).astype(jnp.float32)
        out_refs[li][...] = mask_t.T


def kernel(z, W1_0, b1_0, W2_0, b2_0, W1_1, b1_1, W2_1, b2_1,
           W1_2, b1_2, W2_2, b2_2, W1_3, b1_3, W2_3, b2_3):
    B, Z = z.shape
    R = 1024 if B % 1024 == 0 else B
    grid = (B // R,)
    zt = z.T

    layers = [
        (W1_0, b1_0, W2_0, b2_0),
        (W1_1, b1_1, W2_1, b2_1),
        (W1_2, b1_2, W2_2, b2_2),
        (W1_3, b1_3, W2_3, b2_3),
    ]
    chans = [W2.shape[1] for (_, _, W2, _) in layers]

    in_arrays = [zt]
    in_specs = [pl.BlockSpec((Z, R), lambda i: (0, i))]
    for (W1, b1, W2, b2) in layers:
        for w in (W1.T, b1.reshape(-1, 1), W2.T, b2.reshape(-1, 1)):
            in_arrays.append(w)
            in_specs.append(pl.BlockSpec(w.shape, lambda i: (0, 0)))

    out_shape = [jax.ShapeDtypeStruct((B, c), jnp.float32) for c in chans]
    out_specs = [pl.BlockSpec((R, c), lambda i: (i, 0)) for c in chans]

    out = pl.pallas_call(
        _mask_body,
        grid=grid,
        in_specs=in_specs,
        out_specs=out_specs,
        out_shape=out_shape,
        compiler_params=pltpu.CompilerParams(
            dimension_semantics=("parallel",),
        ),
    )(*in_arrays)
    return tuple(out)


# speedup vs baseline: 1.0147x; 1.0145x over previous
import jax
import jax.numpy as jnp
from jax.experimental import pallas as pl
from jax.experimental.pallas import tpu as pltpu

def _sum0(x):
    n = x.shape[0]
    while n > 1:
        h = n // 2
        x = x[:h] + x[h:]
        n = h
    return x


def _mask_body(zt_ref, *refs):
    in_refs = refs[:16]
    out_refs = refs[16:]
    zt = zt_ref[...]
    r = zt.shape[1]
    his = []
    los = []
    ks = []
    for li in range(4):
        W1t = in_refs[4 * li][...]
        b1 = in_refs[4 * li + 1][...]
        W2t = in_refs[4 * li + 2][...]
        b2 = in_refs[4 * li + 3][...]
        ht = jnp.maximum(
            jnp.dot(W1t, zt, preferred_element_type=jnp.float32) + b1, 0.0)
        logits_t = jnp.dot(W2t, ht, preferred_element_type=jnp.float32) + b2
        u = jax.lax.bitcast_convert_type(logits_t, jnp.int32)
        hi_raw = (u >> 16).astype(jnp.int16)
        sign = jnp.where(hi_raw < 0, jnp.int16(-1), jnp.int16(0))
        his.append(jnp.where(hi_raw >= 0, hi_raw, hi_raw ^ 0x7FFF))
        los.append((u.astype(jnp.int16) ^ sign) ^ (-0x8000))
        ks.append(max(1, logits_t.shape[0] // 2))

    def step_hi(i, ps):
        bit = jnp.broadcast_to(
            jnp.left_shift(jnp.int32(1), jnp.int32(15) - i), (1, r)
        ).astype(jnp.int16)
        out = []
        for hi, k, p in zip(his, ks, ps):
            cand = p | bit
            candb = jnp.broadcast_to(cand ^ (-0x8000), hi.shape)
            cnt = _sum0(jnp.where(hi >= candb, jnp.int16(1), jnp.int16(0)))
            out.append(jnp.where(cnt >= k, cand, p))
        return tuple(out)

    zero16 = tuple(jnp.zeros((1, r), jnp.int16) for _ in range(4))
    ps = jax.lax.fori_loop(0, 16, step_hi, zero16, unroll=8)

    t_his, ms, lims = [], [], []
    for hi, lo, k, p in zip(his, los, ks, ps):
        t_hi = p ^ (-0x8000)
        t_hib = jnp.broadcast_to(t_hi, hi.shape)
        n_gt = _sum0(jnp.where(hi > t_hib, jnp.int16(1), jnp.int16(0)))
        t_his.append(t_hi)
        ms.append(jnp.int16(k) - n_gt)
        lims.append(jnp.where(hi == t_hib, lo, jnp.int16(-0x8000)))

    def step_lo(i, ps2):
        bit = jnp.broadcast_to(
            jnp.left_shift(jnp.int32(1), jnp.int32(15) - i), (1, r)
        ).astype(jnp.int16)
        out = []
        for lom, m, p in zip(lims, ms, ps2):
            cand = p | bit
            candb = jnp.broadcast_to(cand ^ (-0x8000), lom.shape)
            cnt = _sum0(jnp.where(lom >= candb, jnp.int16(1), jnp.int16(0)))
            out.append(jnp.where(cnt >= m, cand, p))
        return tuple(out)

    ps2 = jax.lax.fori_loop(0, 16, step_lo, zero16, unroll=8)

    for li in range(4):
        t_hib = jnp.broadcast_to(t_his[li], his[li].shape)
        t_lob = jnp.broadcast_to(ps2[li] ^ (-0x8000), his[li].shape)
        sel = (his[li] > t_hib) | ((his[li] == t_hib) & (lims[li] >= t_lob))
        mask_t = jnp.where(sel, jnp.bfloat16(2), jnp.bfloat16(0))
        out_refs[li][...] = mask_t.T.astype(jnp.float32)


def kernel(z, W1_0, b1_0, W2_0, b2_0, W1_1, b1_1, W2_1, b2_1,
           W1_2, b1_2, W2_2, b2_2, W1_3, b1_3, W2_3, b2_3):
    B, Z = z.shape
    R = 1024 if B % 1024 == 0 else B
    grid = (B // R,)
    zt = z.T

    layers = [
        (W1_0, b1_0, W2_0, b2_0),
        (W1_1, b1_1, W2_1, b2_1),
        (W1_2, b1_2, W2_2, b2_2),
        (W1_3, b1_3, W2_3, b2_3),
    ]
    chans = [W2.shape[1] for (_, _, W2, _) in layers]

    in_arrays = [zt]
    in_specs = [pl.BlockSpec((Z, R), lambda i: (0, i))]
    for (W1, b1, W2, b2) in layers:
        for w in (W1.T, b1.reshape(-1, 1), W2.T, b2.reshape(-1, 1)):
            in_arrays.append(w)
            in_specs.append(pl.BlockSpec(w.shape, lambda i: (0, 0)))

    out_shape = [jax.ShapeDtypeStruct((B, c), jnp.float32) for c in chans]
    out_specs = [pl.BlockSpec((R, c), lambda i: (i, 0)) for c in chans]

    out = pl.pallas_call(
        _mask_body,
        grid=grid,
        in_specs=in_specs,
        out_specs=out_specs,
        out_shape=out_shape,
        compiler_params=pltpu.CompilerParams(
            dimension_semantics=("parallel",),
        ),
    )(*in_arrays)
    return tuple(out)
